# baseline (speedup 1.0000x reference)
import dataclasses
import functools

import jax
import jax.numpy as jnp
from jax.experimental import pallas as pl
from jax.experimental.pallas import tpu as pltpu
from jax.experimental.pallas import tpu_sc as plsc

N = 10000
E = 320000
F_IN = 128
H = 20
B = 64
C = 10

NW = 32
EPW = E // NW
LANES = 16

FG = 2
NQ = H // FG
NS = 3
E2 = 336384
ET = E2 // NS
CH = 18688
assert ET % CH == 0 and (ET // CH) % 2 == 0 and CH % LANES == 0
assert NQ * NS <= NW
PAD_T = (E2 - E) // NW
MASK14 = (1 << 14) - 1

_MESH = plsc.VectorSubcoreMesh(core_axis_name="c", subcore_axis_name="s")

_SC_CP = pltpu.CompilerParams()
if "needs_layout_passes" in pltpu.CompilerParams.__dataclass_fields__:
    _SC_CP = dataclasses.replace(_SC_CP, needs_layout_passes=False)


def _wid():
    return jax.lax.axis_index("s") * 2 + jax.lax.axis_index("c")


@functools.partial(
    pl.kernel,
    out_type=jax.ShapeDtypeStruct((NW, N), jnp.float32),
    mesh=_MESH,
    scratch_types=[
        pltpu.VMEM((N,), jnp.float32),
        pltpu.VMEM((EPW,), jnp.int32),
        pltpu.VMEM((EPW,), jnp.float32),
    ],
    compiler_params=_SC_CP,
)
def _sc_deg(dst_hbm, ew_hbm, out_hbm, acc, dv, wv):
    wid = _wid()
    base = wid * EPW
    pltpu.sync_copy(dst_hbm.at[pl.ds(base, EPW)], dv)
    pltpu.sync_copy(ew_hbm.at[pl.ds(base, EPW)], wv)

    @plsc.parallel_loop(0, N, LANES, unroll=8)
    def _(i):
        acc[pl.ds(i, LANES)] = jnp.zeros((LANES,), jnp.float32)

    @plsc.parallel_loop(0, EPW, LANES, unroll=8)
    def _(i):
        d = dv[pl.ds(i, LANES)]
        w = wv[pl.ds(i, LANES)]
        plsc.addupdate_scatter(acc, [d], w)

    pltpu.sync_copy(acc, out_hbm.at[wid])


@functools.partial(
    pl.kernel,
    out_type=(
        jax.ShapeDtypeStruct((E2,), jnp.float32),
        jax.ShapeDtypeStruct((E2,), jnp.int32),
    ),
    mesh=_MESH,
    scratch_types=[
        pltpu.VMEM((N,), jnp.float32),
        pltpu.VMEM((EPW,), jnp.int32),
        pltpu.VMEM((EPW,), jnp.int32),
        pltpu.VMEM((EPW,), jnp.float32),
        pltpu.VMEM((EPW,), jnp.float32),
        pltpu.VMEM((EPW,), jnp.int32),
        pltpu.VMEM((PAD_T,), jnp.float32),
        pltpu.VMEM((PAD_T,), jnp.int32),
    ],
    compiler_params=_SC_CP,
)
def _sc_norm(src_hbm, dst_hbm, ew_hbm, dinv_hbm, nrm_hbm, pck_hbm,
             dn, sv, dv, wv, ov, pv, zf, zi):
    wid = _wid()
    base = wid * EPW
    pltpu.sync_copy(dinv_hbm.at[0], dn)
    pltpu.sync_copy(src_hbm.at[pl.ds(base, EPW)], sv)
    pltpu.sync_copy(dst_hbm.at[pl.ds(base, EPW)], dv)
    pltpu.sync_copy(ew_hbm.at[pl.ds(base, EPW)], wv)

    @plsc.parallel_loop(0, EPW, LANES, unroll=8)
    def _(i):
        s = sv[pl.ds(i, LANES)]
        d = dv[pl.ds(i, LANES)]
        w = wv[pl.ds(i, LANES)]
        a = plsc.load_gather(dn, [s])
        b = plsc.load_gather(dn, [d])
        ov[pl.ds(i, LANES)] = a * w * b
        pv[pl.ds(i, LANES)] = jnp.bitwise_or(jax.lax.shift_left(d, 14), s)

    @plsc.parallel_loop(0, PAD_T, LANES, unroll=4)
    def _(i):
        zf[pl.ds(i, LANES)] = jnp.zeros((LANES,), jnp.float32)
        zi[pl.ds(i, LANES)] = jnp.zeros((LANES,), jnp.int32)

    pltpu.sync_copy(ov, nrm_hbm.at[pl.ds(base, EPW)])
    pltpu.sync_copy(pv, pck_hbm.at[pl.ds(base, EPW)])
    pad_base = E + wid * PAD_T
    pltpu.sync_copy(zf, nrm_hbm.at[pl.ds(pad_base, PAD_T)])
    pltpu.sync_copy(zi, pck_hbm.at[pl.ds(pad_base, PAD_T)])


@functools.partial(
    pl.kernel,
    out_type=jax.ShapeDtypeStruct((NS * H, N), jnp.float32),
    mesh=_MESH,
    scratch_types=(
        [pltpu.VMEM((N,), jnp.float32)] * (2 * FG)
        + [
            pltpu.VMEM((CH,), jnp.int32),
            pltpu.VMEM((CH,), jnp.float32),
            pltpu.VMEM((CH,), jnp.int32),
            pltpu.VMEM((CH,), jnp.float32),
            pltpu.SemaphoreType.DMA,
            pltpu.SemaphoreType.DMA,
        ]
    ),
    compiler_params=_SC_CP,
)
def _sc_msg(ht_hbm, pck_hbm, nrm_hbm, out_hbm, *scratch):
    pv0, nv0, pv1, nv1, sem0, sem1 = scratch[2 * FG:]
    wid = _wid()
    hcols = scratch[:FG]
    accs = scratch[FG:2 * FG]

    def start(c, pv, nv, sem):
        pltpu.async_copy(pck_hbm.at[pl.ds(c, CH)], pv, sem)
        pltpu.async_copy(nrm_hbm.at[pl.ds(c, CH)], nv, sem)

    def drain(pv, nv, sem):
        pltpu.make_async_copy(pck_hbm.at[pl.ds(0, CH)], pv, sem).wait()
        pltpu.make_async_copy(nrm_hbm.at[pl.ds(0, CH)], nv, sem).wait()

    def process(pv, nv):
        @plsc.parallel_loop(0, CH, LANES, unroll=8)
        def _(i):
            p = pv[pl.ds(i, LANES)]
            nm = nv[pl.ds(i, LANES)]
            s = jnp.bitwise_and(p, MASK14)
            d = jax.lax.shift_right_logical(p, 14)
            for hc, ac in zip(hcols, accs):
                plsc.addupdate_scatter(ac, [d], plsc.load_gather(hc, [s]) * nm)

    @pl.when(wid < NQ * NS)
    def _():
        quad = wid // NS
        split = wid % NS
        base = split * ET
        start(base, pv0, nv0, sem0)
        for j, hc in enumerate(hcols):
            pltpu.sync_copy(ht_hbm.at[quad * FG + j], hc)

        for ac in accs:
            @plsc.parallel_loop(0, N, LANES, unroll=8)
            def _(i):
                ac[pl.ds(i, LANES)] = jnp.zeros((LANES,), jnp.float32)

        @pl.loop(0, ET, step=2 * CH)
        def _(c):
            start(base + c + CH, pv1, nv1, sem1)
            drain(pv0, nv0, sem0)
            process(pv0, nv0)

            @pl.when(c + 2 * CH < ET)
            def _():
                start(base + c + 2 * CH, pv0, nv0, sem0)

            drain(pv1, nv1, sem1)
            process(pv1, nv1)

        for j, ac in enumerate(accs):
            pltpu.sync_copy(ac, out_hbm.at[split * H + quad * FG + j])


def _tc_prep(deg_parts, x, W1):
    def body(parts_ref, x_ref, w_ref, dinv_ref, ht_ref):
        deg = jnp.sum(parts_ref[...], axis=0, keepdims=True) + 1.0
        dinv_ref[...] = jnp.where(deg > 0, 1.0 / jnp.sqrt(deg), 0.0)
        ht_ref[...] = jax.lax.dot_general(
            w_ref[...], x_ref[...], (((0,), (1,)), ((), ())),
            preferred_element_type=jnp.float32)

    return pl.pallas_call(
        body,
        out_shape=(
            jax.ShapeDtypeStruct((1, N), jnp.float32),
            jax.ShapeDtypeStruct((H, N), jnp.float32),
        ),
    )(deg_parts, x, W1)


def _epilogue(msg_parts, ht, dinv, b_col):
    msg = msg_parts[pl.ds(0, H), :]
    for k in range(1, NS):
        msg = msg + msg_parts[pl.ds(k * H, H), :]
    tmp = msg + dinv * dinv * ht + b_col
    ss = jnp.sum(tmp * tmp, axis=0, keepdims=True)
    nrm = jnp.maximum(jnp.sqrt(ss), 1e-12)
    return jnp.maximum(tmp / nrm, 0.0)


def _tc_mid(msg, ht, dinv, b_col, Wn):
    def body(m_ref, h_ref, di_ref, b_ref, w_ref, o_ref):
        emb = _epilogue(m_ref, h_ref[...], di_ref[...], b_ref[...])
        o_ref[...] = jax.lax.dot_general(
            w_ref[...], emb, (((0,), (0,)), ((), ())),
            preferred_element_type=jnp.float32)

    return pl.pallas_call(
        body,
        out_shape=jax.ShapeDtypeStruct((H, N), jnp.float32),
    )(msg, ht, dinv, b_col, Wn)


def _tc_final(msg, ht, dinv, b_col, batch2d, Wl, bl):
    def body(m_ref, h_ref, di_ref, b_ref, bt_ref, wl_ref, bl_ref, o_ref,
             feats_ref):
        emb = _epilogue(m_ref, h_ref[...], di_ref[...], b_ref[...])
        bt = bt_ref[...]
        gids = jax.lax.broadcasted_iota(jnp.int32, (B, 1), 0)
        onehot = (bt == gids).astype(jnp.float32)
        cnt = jnp.sum(onehot, axis=1)
        gsum_t = jax.lax.dot_general(
            emb, onehot, (((1,), (1,)), ((), ())),
            preferred_element_type=jnp.float32)
        gmean_t = gsum_t / jnp.maximum(cnt, 1.0)[None, :]
        for g in range(B):
            mg = jnp.where(bt == g, emb, 0.0)
            feats_ref[g, pl.ds(0, H)] = jnp.max(mg, axis=1)
            feats_ref[g, pl.ds(H, H)] = gmean_t[:, g]
        o_ref[...] = jnp.dot(feats_ref[...], wl_ref[...],
                             preferred_element_type=jnp.float32) + bl_ref[...]

    return pl.pallas_call(
        body,
        out_shape=jax.ShapeDtypeStruct((B, C), jnp.float32),
        scratch_shapes=[pltpu.VMEM((B, 2 * H), jnp.float32)],
    )(msg, ht, dinv, b_col, batch2d, Wl, bl)


def kernel(x, edge_index, batch, edge_weights, W1, b1, W2, b2, W3, b3, Wl, bl):
    src = edge_index[0]
    dst = edge_index[1]
    batch2d = batch.reshape(1, N)

    deg_parts = _sc_deg(dst, edge_weights)
    dinv, ht1 = _tc_prep(deg_parts, x, W1)
    norm, packed = _sc_norm(src, dst, edge_weights, dinv)

    msg1 = _sc_msg(ht1, packed, norm)
    ht2 = _tc_mid(msg1, ht1, dinv, b1.reshape(H, 1), W2)
    msg2 = _sc_msg(ht2, packed, norm)
    ht3 = _tc_mid(msg2, ht2, dinv, b2.reshape(H, 1), W3)
    msg3 = _sc_msg(ht3, packed, norm)
    return _tc_final(msg3, ht3, dinv, b3.reshape(H, 1), batch2d, Wl, bl)


# speedup vs baseline: 1.0311x; 1.0311x over previous
import dataclasses
import functools

import jax
import jax.numpy as jnp
from jax.experimental import pallas as pl
from jax.experimental.pallas import tpu as pltpu
from jax.experimental.pallas import tpu_sc as plsc

N = 10000
E = 320000
F_IN = 128
H = 20
B = 64
C = 10

NW = 32
EPW = E // NW
LANES = 16

FG = 2
NQ = H // FG
NS = 3
E2 = 336384
ET = E2 // NS
CH = 9344
assert ET % CH == 0 and (ET // CH) % 2 == 0 and CH % LANES == 0
assert NQ * NS <= NW
PAD_T = (E2 - E) // NW
MASK14 = (1 << 14) - 1

_MESH = plsc.VectorSubcoreMesh(core_axis_name="c", subcore_axis_name="s")

_SC_CP = pltpu.CompilerParams()
if "needs_layout_passes" in pltpu.CompilerParams.__dataclass_fields__:
    _SC_CP = dataclasses.replace(_SC_CP, needs_layout_passes=False)


def _wid():
    return jax.lax.axis_index("s") * 2 + jax.lax.axis_index("c")


@functools.partial(
    pl.kernel,
    out_type=jax.ShapeDtypeStruct((NW, N), jnp.float32),
    mesh=_MESH,
    scratch_types=[
        pltpu.VMEM((N,), jnp.float32),
        pltpu.VMEM((EPW,), jnp.int32),
        pltpu.VMEM((EPW,), jnp.float32),
    ],
    compiler_params=_SC_CP,
)
def _sc_deg(dst_hbm, ew_hbm, out_hbm, acc, dv, wv):
    wid = _wid()
    base = wid * EPW
    pltpu.sync_copy(dst_hbm.at[pl.ds(base, EPW)], dv)
    pltpu.sync_copy(ew_hbm.at[pl.ds(base, EPW)], wv)

    @plsc.parallel_loop(0, N, LANES, unroll=8)
    def _(i):
        acc[pl.ds(i, LANES)] = jnp.zeros((LANES,), jnp.float32)

    @plsc.parallel_loop(0, EPW, LANES, unroll=8)
    def _(i):
        d = dv[pl.ds(i, LANES)]
        w = wv[pl.ds(i, LANES)]
        plsc.addupdate_scatter(acc, [d], w)

    pltpu.sync_copy(acc, out_hbm.at[wid])


@functools.partial(
    pl.kernel,
    out_type=(
        jax.ShapeDtypeStruct((E2,), jnp.float32),
        jax.ShapeDtypeStruct((E2,), jnp.int32),
    ),
    mesh=_MESH,
    scratch_types=[
        pltpu.VMEM((N,), jnp.float32),
        pltpu.VMEM((EPW,), jnp.int32),
        pltpu.VMEM((EPW,), jnp.int32),
        pltpu.VMEM((EPW,), jnp.float32),
        pltpu.VMEM((EPW,), jnp.float32),
        pltpu.VMEM((EPW,), jnp.int32),
        pltpu.VMEM((PAD_T,), jnp.float32),
        pltpu.VMEM((PAD_T,), jnp.int32),
    ],
    compiler_params=_SC_CP,
)
def _sc_norm(src_hbm, dst_hbm, ew_hbm, dinv_hbm, nrm_hbm, pck_hbm,
             dn, sv, dv, wv, ov, pv, zf, zi):
    wid = _wid()
    base = wid * EPW
    pltpu.sync_copy(dinv_hbm.at[0], dn)
    pltpu.sync_copy(src_hbm.at[pl.ds(base, EPW)], sv)
    pltpu.sync_copy(dst_hbm.at[pl.ds(base, EPW)], dv)
    pltpu.sync_copy(ew_hbm.at[pl.ds(base, EPW)], wv)

    @plsc.parallel_loop(0, EPW, LANES, unroll=8)
    def _(i):
        s = sv[pl.ds(i, LANES)]
        d = dv[pl.ds(i, LANES)]
        w = wv[pl.ds(i, LANES)]
        a = plsc.load_gather(dn, [s])
        b = plsc.load_gather(dn, [d])
        ov[pl.ds(i, LANES)] = a * w * b
        pv[pl.ds(i, LANES)] = jnp.bitwise_or(jax.lax.shift_left(d, 14), s)

    @plsc.parallel_loop(0, PAD_T, LANES, unroll=4)
    def _(i):
        zf[pl.ds(i, LANES)] = jnp.zeros((LANES,), jnp.float32)
        zi[pl.ds(i, LANES)] = jnp.zeros((LANES,), jnp.int32)

    pltpu.sync_copy(ov, nrm_hbm.at[pl.ds(base, EPW)])
    pltpu.sync_copy(pv, pck_hbm.at[pl.ds(base, EPW)])
    pad_base = E + wid * PAD_T
    pltpu.sync_copy(zf, nrm_hbm.at[pl.ds(pad_base, PAD_T)])
    pltpu.sync_copy(zi, pck_hbm.at[pl.ds(pad_base, PAD_T)])


@functools.partial(
    pl.kernel,
    out_type=jax.ShapeDtypeStruct((NS * H, N), jnp.float32),
    mesh=_MESH,
    scratch_types=(
        [pltpu.VMEM((N,), jnp.float32)] * (2 * FG)
        + [
            pltpu.VMEM((CH,), jnp.int32),
            pltpu.VMEM((CH,), jnp.float32),
            pltpu.VMEM((CH,), jnp.int32),
            pltpu.VMEM((CH,), jnp.float32),
            pltpu.SemaphoreType.DMA,
            pltpu.SemaphoreType.DMA,
            pltpu.SemaphoreType.DMA,
        ]
    ),
    compiler_params=_SC_CP,
)
def _sc_msg(ht_hbm, pck_hbm, nrm_hbm, out_hbm, *scratch):
    pv0, nv0, pv1, nv1, sem0, sem1, sem2 = scratch[2 * FG:]
    wid = _wid()
    hcols = scratch[:FG]
    accs = scratch[FG:2 * FG]

    def start(c, pv, nv, sem):
        pltpu.async_copy(pck_hbm.at[pl.ds(c, CH)], pv, sem)
        pltpu.async_copy(nrm_hbm.at[pl.ds(c, CH)], nv, sem)

    def drain(pv, nv, sem):
        pltpu.make_async_copy(pck_hbm.at[pl.ds(0, CH)], pv, sem).wait()
        pltpu.make_async_copy(nrm_hbm.at[pl.ds(0, CH)], nv, sem).wait()

    def process(pv, nv):
        @plsc.parallel_loop(0, CH, LANES, unroll=8 // FG)
        def _(i):
            p = pv[pl.ds(i, LANES)]
            nm = nv[pl.ds(i, LANES)]
            s = jnp.bitwise_and(p, MASK14)
            d = jax.lax.shift_right_logical(p, 14)
            for hc, ac in zip(hcols, accs):
                plsc.addupdate_scatter(ac, [d], plsc.load_gather(hc, [s]) * nm)

    @pl.when(wid < NQ * NS)
    def _():
        quad = wid // NS
        split = wid % NS
        base = split * ET
        start(base, pv0, nv0, sem0)
        for j, hc in enumerate(hcols):
            pltpu.async_copy(ht_hbm.at[quad * FG + j], hc, sem2)

        for ac in accs:
            @plsc.parallel_loop(0, N, LANES, unroll=8)
            def _(i):
                ac[pl.ds(i, LANES)] = jnp.zeros((LANES,), jnp.float32)

        for j, hc in enumerate(hcols):
            pltpu.make_async_copy(ht_hbm.at[quad * FG + j], hc, sem2).wait()

        @pl.loop(0, ET, step=2 * CH)
        def _(c):
            start(base + c + CH, pv1, nv1, sem1)
            drain(pv0, nv0, sem0)
            process(pv0, nv0)

            @pl.when(c + 2 * CH < ET)
            def _():
                start(base + c + 2 * CH, pv0, nv0, sem0)

            drain(pv1, nv1, sem1)
            process(pv1, nv1)

        for j, ac in enumerate(accs):
            pltpu.async_copy(ac, out_hbm.at[split * H + quad * FG + j], sem2)
        for j, ac in enumerate(accs):
            pltpu.make_async_copy(ac, out_hbm.at[split * H + quad * FG + j],
                                  sem2).wait()


def _tc_prep(deg_parts, x, W1):
    def body(parts_ref, x_ref, w_ref, dinv_ref, ht_ref):
        deg = jnp.sum(parts_ref[...], axis=0, keepdims=True) + 1.0
        dinv_ref[...] = jnp.where(deg > 0, 1.0 / jnp.sqrt(deg), 0.0)
        ht_ref[...] = jax.lax.dot_general(
            w_ref[...], x_ref[...], (((0,), (1,)), ((), ())),
            preferred_element_type=jnp.float32)

    return pl.pallas_call(
        body,
        out_shape=(
            jax.ShapeDtypeStruct((1, N), jnp.float32),
            jax.ShapeDtypeStruct((H, N), jnp.float32),
        ),
    )(deg_parts, x, W1)


def _epilogue(msg_parts, ht, dinv, b_col):
    msg = msg_parts[pl.ds(0, H), :]
    for k in range(1, NS):
        msg = msg + msg_parts[pl.ds(k * H, H), :]
    tmp = msg + dinv * dinv * ht + b_col
    ss = jnp.sum(tmp * tmp, axis=0, keepdims=True)
    nrm = jnp.maximum(jnp.sqrt(ss), 1e-12)
    return jnp.maximum(tmp / nrm, 0.0)


def _tc_mid(msg, ht, dinv, b_col, Wn):
    def body(m_ref, h_ref, di_ref, b_ref, w_ref, o_ref):
        emb = _epilogue(m_ref, h_ref[...], di_ref[...], b_ref[...])
        o_ref[...] = jax.lax.dot_general(
            w_ref[...], emb, (((0,), (0,)), ((), ())),
            preferred_element_type=jnp.float32)

    return pl.pallas_call(
        body,
        out_shape=jax.ShapeDtypeStruct((H, N), jnp.float32),
    )(msg, ht, dinv, b_col, Wn)


def _tc_final(msg, ht, dinv, b_col, batch2d, Wl, bl):
    def body(m_ref, h_ref, di_ref, b_ref, bt_ref, wl_ref, bl_ref, o_ref,
             feats_ref):
        emb = _epilogue(m_ref, h_ref[...], di_ref[...], b_ref[...])
        bt = bt_ref[...]
        gids = jax.lax.broadcasted_iota(jnp.int32, (B, 1), 0)
        onehot = (bt == gids).astype(jnp.float32)
        cnt = jnp.sum(onehot, axis=1)
        gsum_t = jax.lax.dot_general(
            emb, onehot, (((1,), (1,)), ((), ())),
            preferred_element_type=jnp.float32)
        gmean_t = gsum_t / jnp.maximum(cnt, 1.0)[None, :]
        for g in range(B):
            mg = jnp.where(bt == g, emb, 0.0)
            feats_ref[g, pl.ds(0, H)] = jnp.max(mg, axis=1)
            feats_ref[g, pl.ds(H, H)] = gmean_t[:, g]
        o_ref[...] = jnp.dot(feats_ref[...], wl_ref[...],
                             preferred_element_type=jnp.float32) + bl_ref[...]

    return pl.pallas_call(
        body,
        out_shape=jax.ShapeDtypeStruct((B, C), jnp.float32),
        scratch_shapes=[pltpu.VMEM((B, 2 * H), jnp.float32)],
    )(msg, ht, dinv, b_col, batch2d, Wl, bl)


def kernel(x, edge_index, batch, edge_weights, W1, b1, W2, b2, W3, b3, Wl, bl):
    src = edge_index[0]
    dst = edge_index[1]
    batch2d = batch.reshape(1, N)

    deg_parts = _sc_deg(dst, edge_weights)
    dinv, ht1 = _tc_prep(deg_parts, x, W1)
    norm, packed = _sc_norm(src, dst, edge_weights, dinv)

    msg1 = _sc_msg(ht1, packed, norm)
    ht2 = _tc_mid(msg1, ht1, dinv, b1.reshape(H, 1), W2)
    msg2 = _sc_msg(ht2, packed, norm)
    ht3 = _tc_mid(msg2, ht2, dinv, b2.reshape(H, 1), W3)
    msg3 = _sc_msg(ht3, packed, norm)
    return _tc_final(msg3, ht3, dinv, b3.reshape(H, 1), batch2d, Wl, bl)
